# baseline (speedup 1.0000x reference)
import functools

import jax
import jax.numpy as jnp
from jax import lax
from jax.experimental import pallas as pl
from jax.experimental.pallas import tpu as pltpu
from jax.experimental.pallas import tpu_sc as plsc

_C = 128
_K = 4


@functools.lru_cache(maxsize=None)
def _build(n_total, n_vocab, d):
    info = plsc.get_sparse_core_info()
    num_cores, num_subcores = info.num_cores, info.num_subcores
    num_workers = num_cores * num_subcores
    rows_per_slot = _C * _K
    n_per_w = n_total // num_workers
    assert n_total % num_workers == 0
    assert n_per_w % rows_per_slot == 0
    n_slots = n_per_w // rows_per_slot

    mesh = plsc.VectorSubcoreMesh(core_axis_name="c", subcore_axis_name="s")

    @functools.partial(
        pl.kernel,
        mesh=mesh,
        out_type=jax.ShapeDtypeStruct((n_total, d), jnp.float32),
        scratch_types=[
            pltpu.VMEM((_K, _C), jnp.int32),
            pltpu.VMEM((rows_per_slot, d), jnp.float32),
            pltpu.SemaphoreType.DMA,
        ],
        compiler_params=pltpu.CompilerParams(use_tc_tiling_on_sc=False),
    )
    def k(idx_hbm, tab_hbm, out_hbm, idx_v, rows_v, gsem):
        wid = lax.axis_index("s") * num_cores + lax.axis_index("c")
        base_row = wid * (n_per_w // _C)
        base_out = wid * n_per_w

        def body(g, _):
            row0 = base_row + g * _K
            pltpu.sync_copy(idx_hbm.at[pl.ds(row0, _K)], idx_v)
            for j in range(_K):
                pltpu.async_copy(
                    tab_hbm.at[idx_v.at[j]],
                    rows_v.at[pl.ds(j * _C, _C)],
                    gsem,
                )
            for j in range(_K):
                pltpu.make_async_copy(
                    tab_hbm.at[idx_v.at[j]],
                    rows_v.at[pl.ds(j * _C, _C)],
                    gsem,
                ).wait()
            pltpu.sync_copy(
                rows_v, out_hbm.at[pl.ds(base_out + g * rows_per_slot, rows_per_slot)]
            )
            return ()

        lax.fori_loop(0, n_slots, body, ())

    return k


def kernel(indices, table):
    b, h = indices.shape
    v, d = table.shape
    n_total = b * h
    idx2 = indices.reshape(n_total // _C, _C).astype(jnp.int32)
    out = _build(n_total, v, d)(idx2, table)
    return out.reshape(b, h, d)


# speedup vs baseline: 1.0711x; 1.0711x over previous
import functools

import jax
import jax.numpy as jnp
from jax import lax
from jax.experimental import pallas as pl
from jax.experimental.pallas import tpu as pltpu
from jax.experimental.pallas import tpu_sc as plsc

_C = 128
_K = 2
_NBUF = 4


@functools.lru_cache(maxsize=None)
def _build(n_total, n_vocab, d):
    info = plsc.get_sparse_core_info()
    num_cores, num_subcores = info.num_cores, info.num_subcores
    num_workers = num_cores * num_subcores
    rows_per_slot = _C * _K
    n_per_w = n_total // num_workers
    assert n_total % num_workers == 0
    assert n_per_w % rows_per_slot == 0
    n_slots = n_per_w // rows_per_slot
    assert n_slots % _NBUF == 0
    n_outer = n_slots // _NBUF

    mesh = plsc.VectorSubcoreMesh(core_axis_name="c", subcore_axis_name="s")

    @functools.partial(
        pl.kernel,
        mesh=mesh,
        out_type=jax.ShapeDtypeStruct((n_total, d), jnp.float32),
        scratch_types=[
            pltpu.VMEM((_NBUF, _K, _C), jnp.int32),
            pltpu.VMEM((_NBUF, rows_per_slot, d), jnp.float32),
            [pltpu.SemaphoreType.DMA] * _NBUF,
            [pltpu.SemaphoreType.DMA] * _NBUF,
        ],
        compiler_params=pltpu.CompilerParams(use_tc_tiling_on_sc=False),
    )
    def k(idx_hbm, tab_hbm, out_hbm, idx_v, rows_v, gsem, osem):
        wid = lax.axis_index("s") * num_cores + lax.axis_index("c")
        base_row = wid * (n_per_w // _C)
        base_out = wid * n_per_w

        def fire(slot, g):
            pltpu.sync_copy(idx_hbm.at[pl.ds(base_row + g * _K, _K)], idx_v.at[slot])
            for j in range(_K):
                pltpu.async_copy(
                    tab_hbm.at[idx_v.at[slot, j]],
                    rows_v.at[slot, pl.ds(j * _C, _C)],
                    gsem[slot],
                )

        def drain(slot):
            for j in range(_K):
                pltpu.make_async_copy(
                    tab_hbm.at[idx_v.at[slot, j]],
                    rows_v.at[slot, pl.ds(j * _C, _C)],
                    gsem[slot],
                ).wait()

        def out_copy(slot, g):
            return pltpu.async_copy(
                rows_v.at[slot],
                out_hbm.at[pl.ds(base_out + g * rows_per_slot, rows_per_slot)],
                osem[slot],
            )

        def wait_out(slot, g):
            pltpu.make_async_copy(
                rows_v.at[slot],
                out_hbm.at[pl.ds(base_out + g * rows_per_slot, rows_per_slot)],
                osem[slot],
            ).wait()

        fire(0, 0)
        fire(1, 1)

        def outer(go, _):
            for b in range(_NBUF):
                g = go * _NBUF + b
                fslot = (b + 2) % _NBUF
                if b < 2:
                    @pl.when(go >= 1)
                    def _():
                        wait_out(fslot, g - 2)
                else:
                    wait_out(fslot, g - 2)
                if b < 2:
                    fire(fslot, g + 2)
                else:
                    @pl.when(go < n_outer - 1)
                    def _():
                        fire(fslot, g + 2)
                drain(b)
                out_copy(b, g)
            return ()

        lax.fori_loop(0, n_outer, outer, ())

        wait_out((n_slots - 2) % _NBUF, n_slots - 2)
        wait_out((n_slots - 1) % _NBUF, n_slots - 1)

    return k


def kernel(indices, table):
    b, h = indices.shape
    v, d = table.shape
    n_total = b * h
    idx2 = indices.reshape(n_total // _C, _C).astype(jnp.int32)
    out = _build(n_total, v, d)(idx2, table)
    return out.reshape(b, h, d)


# speedup vs baseline: 1.0745x; 1.0032x over previous
import functools

import jax
import jax.numpy as jnp
from jax import lax
from jax.experimental import pallas as pl
from jax.experimental.pallas import tpu as pltpu
from jax.experimental.pallas import tpu_sc as plsc

_R = 2
_NBUF = 4
_CMAX = 128


@functools.lru_cache(maxsize=None)
def _build(batch, hist, n_vocab, d):
    info = plsc.get_sparse_core_info()
    num_cores, num_subcores = info.num_cores, info.num_subcores
    num_workers = num_cores * num_subcores
    rows_per_w = batch // num_workers
    assert batch % num_workers == 0
    assert rows_per_w % (_R * _NBUF) == 0
    n_slots = rows_per_w // _R
    n_outer = n_slots // _NBUF
    chunks = []
    off = 0
    while off < hist:
        c = min(_CMAX, hist - off)
        chunks.append((off, c))
        off += c

    mesh = plsc.VectorSubcoreMesh(core_axis_name="c", subcore_axis_name="s")

    @functools.partial(
        pl.kernel,
        mesh=mesh,
        out_type=jax.ShapeDtypeStruct((batch, hist, d), jnp.float32),
        scratch_types=[
            pltpu.VMEM((_NBUF, _R, hist), jnp.int32),
            pltpu.VMEM((_NBUF, _R, hist, d), jnp.float32),
            [pltpu.SemaphoreType.DMA] * _NBUF,
            [pltpu.SemaphoreType.DMA] * _NBUF,
        ],
        compiler_params=pltpu.CompilerParams(use_tc_tiling_on_sc=False),
    )
    def k(idx_hbm, tab_hbm, out_hbm, idx_v, rows_v, gsem, osem):
        wid = lax.axis_index("s") * num_cores + lax.axis_index("c")
        base_b = wid * rows_per_w

        def fire(slot, g):
            b0 = base_b + g * _R
            pltpu.sync_copy(idx_hbm.at[pl.ds(b0, _R)], idx_v.at[slot])
            for r in range(_R):
                for off, c in chunks:
                    pltpu.async_copy(
                        tab_hbm.at[idx_v.at[slot, r, pl.ds(off, c)]],
                        rows_v.at[slot, r, pl.ds(off, c)],
                        gsem[slot],
                    )

        def drain(slot):
            for r in range(_R):
                for off, c in chunks:
                    pltpu.make_async_copy(
                        tab_hbm.at[idx_v.at[slot, r, pl.ds(off, c)]],
                        rows_v.at[slot, r, pl.ds(off, c)],
                        gsem[slot],
                    ).wait()

        def out_copy(slot, g):
            pltpu.async_copy(
                rows_v.at[slot],
                out_hbm.at[pl.ds(base_b + g * _R, _R)],
                osem[slot],
            )

        def wait_out(slot, g):
            pltpu.make_async_copy(
                rows_v.at[slot],
                out_hbm.at[pl.ds(base_b + g * _R, _R)],
                osem[slot],
            ).wait()

        fire(0, 0)
        fire(1, 1)

        def outer(go, _):
            for b in range(_NBUF):
                g = go * _NBUF + b
                fslot = (b + 2) % _NBUF
                if b < 2:
                    @pl.when(go >= 1)
                    def _():
                        wait_out(fslot, g - 2)
                else:
                    wait_out(fslot, g - 2)
                if b < 2:
                    fire(fslot, g + 2)
                else:
                    @pl.when(go < n_outer - 1)
                    def _():
                        fire(fslot, g + 2)
                drain(b)
                out_copy(b, g)
            return ()

        lax.fori_loop(0, n_outer, outer, ())

        wait_out((n_slots - 2) % _NBUF, n_slots - 2)
        wait_out((n_slots - 1) % _NBUF, n_slots - 1)

    return k


def kernel(indices, table):
    b, h = indices.shape
    v, d = table.shape
    return _build(b, h, v, d)(indices.astype(jnp.int32), table)


# speedup vs baseline: 1.7746x; 1.6515x over previous
import functools

import jax
import jax.numpy as jnp
from jax import lax
from jax.experimental import pallas as pl
from jax.experimental.pallas import tpu as pltpu
from jax.experimental.pallas import tpu_sc as plsc

_R = 2
_NBUF = 4
_CMAX = 128


@functools.lru_cache(maxsize=None)
def _build(batch, hist, n_vocab, d):
    info = plsc.get_sparse_core_info()
    num_cores, num_subcores = info.num_cores, info.num_subcores
    num_workers = num_cores * num_subcores
    rows_per_w = batch // num_workers
    assert batch % num_workers == 0
    assert rows_per_w % (_R * _NBUF) == 0
    n_slots = rows_per_w // _R
    n_outer = n_slots // _NBUF
    chunks = []
    off = 0
    while off < hist:
        c = min(_CMAX, hist - off)
        chunks.append((off, c))
        off += c

    mesh = plsc.VectorSubcoreMesh(core_axis_name="c", subcore_axis_name="s")

    @functools.partial(
        pl.kernel,
        mesh=mesh,
        out_type=jax.ShapeDtypeStruct((batch * hist, 2 * d), jnp.float32),
        scratch_types=[
            pltpu.VMEM((_NBUF, _R, hist), jnp.int32),
            pltpu.VMEM((_NBUF, _R * hist, d), jnp.float32),
            [pltpu.SemaphoreType.DMA] * _NBUF,
            [pltpu.SemaphoreType.DMA] * _NBUF,
        ],
        compiler_params=pltpu.CompilerParams(use_tc_tiling_on_sc=False),
    )
    def k(idx_hbm, tab_hbm, out_hbm, idx_v, rows_v, gsem, osem):
        wid = lax.axis_index("s") * num_cores + lax.axis_index("c")
        base_b = wid * rows_per_w

        def fire(slot, g):
            b0 = base_b + g * _R
            pltpu.sync_copy(idx_hbm.at[pl.ds(b0, _R)], idx_v.at[slot])
            for r in range(_R):
                for off, c in chunks:
                    pltpu.async_copy(
                        tab_hbm.at[idx_v.at[slot, r, pl.ds(off, c)]],
                        rows_v.at[slot, pl.ds(r * hist + off, c)],
                        gsem[slot],
                    )

        def drain(slot):
            for r in range(_R):
                for off, c in chunks:
                    pltpu.make_async_copy(
                        tab_hbm.at[idx_v.at[slot, r, pl.ds(off, c)]],
                        rows_v.at[slot, pl.ds(r * hist + off, c)],
                        gsem[slot],
                    ).wait()

        def out_copy(slot, g):
            pltpu.async_copy(
                rows_v.at[slot],
                out_hbm.at[pl.ds((base_b + g * _R) * hist, _R * hist), pl.ds(0, d)],
                osem[slot],
            )

        def wait_out(slot, g):
            pltpu.make_async_copy(
                rows_v.at[slot],
                out_hbm.at[pl.ds((base_b + g * _R) * hist, _R * hist), pl.ds(0, d)],
                osem[slot],
            ).wait()

        fire(0, 0)
        fire(1, 1)

        def outer(go, _):
            for b in range(_NBUF):
                g = go * _NBUF + b
                fslot = (b + 2) % _NBUF
                if b < 2:
                    @pl.when(go >= 1)
                    def _():
                        wait_out(fslot, g - 2)
                else:
                    wait_out(fslot, g - 2)
                if b < 2:
                    fire(fslot, g + 2)
                else:
                    @pl.when(go < n_outer - 1)
                    def _():
                        fire(fslot, g + 2)
                drain(b)
                out_copy(b, g)
            return ()

        lax.fori_loop(0, n_outer, outer, ())

        wait_out((n_slots - 2) % _NBUF, n_slots - 2)
        wait_out((n_slots - 1) % _NBUF, n_slots - 1)

    return k


def kernel(indices, table):
    b, h = indices.shape
    v, d = table.shape
    out = _build(b, h, v, d)(indices.astype(jnp.int32), table)
    return out[:, :d].reshape(b, h, d)
